# baseline (speedup 1.0000x reference)
import functools

import jax
import jax.numpy as jnp
from jax import lax
from jax.experimental import pallas as pl
from jax.experimental.pallas import tpu as pltpu
from jax.experimental.pallas import tpu_sc as plsc

_EPS = 0.1
_B, _H, _D = 8, 512, 512
_N = _B * _H * _D
_NC, _NS = 2, 16
_NW = _NC * _NS
_PER_W = _N // _NW
_CHUNK = 8192
_NCHUNK = _PER_W // _CHUNK
_LANES = 16
_UNROLL = 16


_ROWS_PER_CHUNK = _CHUNK // _D
_ROWS_PER_TILE = _PER_W // _D
_TILES_PER_B = _H // _ROWS_PER_TILE


def _reduce_chunk(buf, accs):
    def body(i, carry):
        a0, a1, a2, a3 = carry
        for u in range(0, _D // _LANES, 4):
            a0 = a0 + buf[i, pl.ds((u + 0) * _LANES, _LANES)]
            a1 = a1 + buf[i, pl.ds((u + 1) * _LANES, _LANES)]
            a2 = a2 + buf[i, pl.ds((u + 2) * _LANES, _LANES)]
            a3 = a3 + buf[i, pl.ds((u + 3) * _LANES, _LANES)]
        return (a0, a1, a2, a3)

    return lax.fori_loop(0, _ROWS_PER_CHUNK, body, accs)


def _partial_sums(gt):
    mesh = plsc.VectorSubcoreMesh(core_axis_name="c", subcore_axis_name="s")

    @functools.partial(
        pl.kernel,
        out_type=jax.ShapeDtypeStruct((_NW, _LANES), jnp.int32),
        mesh=mesh,
        scratch_types=[
            pltpu.VMEM((2, _ROWS_PER_CHUNK, _D), jnp.int32),
            pltpu.VMEM((_LANES,), jnp.int32),
            pltpu.SemaphoreType.DMA,
            pltpu.SemaphoreType.DMA,
        ],
    )
    def k(x_hbm, out_hbm, buf, stage, sem0, sem1):
        c = lax.axis_index("c")
        s = lax.axis_index("s")
        wid = s * _NC + c
        b = wid // _TILES_PER_B
        r0 = (wid % _TILES_PER_B) * _ROWS_PER_TILE

        sems = (sem0, sem1)
        zero = jnp.zeros((_LANES,), jnp.int32)

        def chunk_rows(j):
            return r0 + lax.rem(j, _NCHUNK) * _ROWS_PER_CHUNK

        pltpu.async_copy(
            x_hbm.at[b, pl.ds(chunk_rows(0), _ROWS_PER_CHUNK)],
            buf.at[0], sems[0])
        pltpu.async_copy(
            x_hbm.at[b, pl.ds(chunk_rows(1), _ROWS_PER_CHUNK)],
            buf.at[1], sems[1])

        def outer(g, accs):
            for bslot in range(2):
                j = g * 2 + bslot
                pltpu.make_async_copy(
                    x_hbm.at[b, pl.ds(chunk_rows(j), _ROWS_PER_CHUNK)],
                    buf.at[bslot], sems[bslot]).wait()
                accs = _reduce_chunk(buf.at[bslot], accs)
                pltpu.async_copy(
                    x_hbm.at[b, pl.ds(chunk_rows(j + 2), _ROWS_PER_CHUNK)],
                    buf.at[bslot], sems[bslot])
            return accs

        accs = lax.fori_loop(
            0, _NCHUNK // 2, outer, (zero, zero, zero, zero))

        for bslot in range(2):
            pltpu.make_async_copy(
                x_hbm.at[b, pl.ds(r0, _ROWS_PER_CHUNK)],
                buf.at[bslot], sems[bslot]).wait()

        stage[...] = (accs[0] + accs[1]) + (accs[2] + accs[3])
        pltpu.sync_copy(stage, out_hbm.at[wid])

    return k(gt)


def kernel(gt):
    partials = _partial_sums(gt)
    count1 = jnp.sum(partials)
    sizes = jnp.stack(
        [jnp.int32(_N) - count1, count1]).astype(jnp.float32)
    lowbound = (sizes * (1.0 - _EPS)).astype(jnp.int32)
    highbound = (sizes * (1.0 + _EPS)).astype(jnp.int32)
    return jnp.stack((lowbound, highbound))


# speedup vs baseline: 1.0357x; 1.0357x over previous
import functools

import jax
import jax.numpy as jnp
from jax import lax
from jax.experimental import pallas as pl
from jax.experimental.pallas import tpu as pltpu
from jax.experimental.pallas import tpu_sc as plsc

_EPS = 0.1
_B, _H, _D = 8, 512, 512
_N = _B * _H * _D
_NC, _NS = 2, 16
_NW = _NC * _NS
_PER_W = _N // _NW
_CHUNK = 8192
_NCHUNK = _PER_W // _CHUNK
_LANES = 16
_UNROLL = 16


_ROWS_PER_CHUNK = _CHUNK // _D
_ROWS_PER_TILE = _PER_W // _D
_TILES_PER_B = _H // _ROWS_PER_TILE


def _reduce_chunk(buf, accs):
    def body(i, carry):
        a0, a1, a2, a3 = carry
        for u in range(0, _D // _LANES, 4):
            a0 = a0 + buf[i, pl.ds((u + 0) * _LANES, _LANES)]
            a1 = a1 + buf[i, pl.ds((u + 1) * _LANES, _LANES)]
            a2 = a2 + buf[i, pl.ds((u + 2) * _LANES, _LANES)]
            a3 = a3 + buf[i, pl.ds((u + 3) * _LANES, _LANES)]
        return (a0, a1, a2, a3)

    return lax.fori_loop(0, _ROWS_PER_CHUNK, body, accs)


def _partial_sums(gt):
    mesh = plsc.VectorSubcoreMesh(core_axis_name="c", subcore_axis_name="s")

    @functools.partial(
        pl.kernel,
        out_type=jax.ShapeDtypeStruct((_NW, _LANES), jnp.int32),
        mesh=mesh,
        scratch_types=[
            pltpu.VMEM((4, _ROWS_PER_CHUNK, _D), jnp.int32),
            pltpu.VMEM((_LANES,), jnp.int32),
            pltpu.SemaphoreType.DMA,
            pltpu.SemaphoreType.DMA,
            pltpu.SemaphoreType.DMA,
            pltpu.SemaphoreType.DMA,
        ],
    )
    def k(x_hbm, out_hbm, buf, stage, sem0, sem1, sem2, sem3):
        c = lax.axis_index("c")
        s = lax.axis_index("s")
        wid = s * _NC + c
        b = wid // _TILES_PER_B
        r0 = (wid % _TILES_PER_B) * _ROWS_PER_TILE

        sems = (sem0, sem1, sem2, sem3)
        zero = jnp.zeros((_LANES,), jnp.int32)
        accs = (zero, zero, zero, zero)

        copies = [None] * _NCHUNK
        for j in range(3):
            copies[j] = pltpu.async_copy(
                x_hbm.at[b, pl.ds(r0 + j * _ROWS_PER_CHUNK,
                                  _ROWS_PER_CHUNK)],
                buf.at[j], sems[j])
        for j in range(_NCHUNK):
            if j + 3 < _NCHUNK:
                copies[j + 3] = pltpu.async_copy(
                    x_hbm.at[b, pl.ds(r0 + (j + 3) * _ROWS_PER_CHUNK,
                                      _ROWS_PER_CHUNK)],
                    buf.at[(j + 3) % 4],
                    sems[(j + 3) % 4],
                )
            copies[j].wait()
            accs = _reduce_chunk(buf.at[j % 4], accs)

        stage[...] = (accs[0] + accs[1]) + (accs[2] + accs[3])
        pltpu.sync_copy(stage, out_hbm.at[wid])

    return k(gt)


def kernel(gt):
    partials = _partial_sums(gt)
    count1 = jnp.sum(partials)
    sizes = jnp.stack(
        [jnp.int32(_N) - count1, count1]).astype(jnp.float32)
    lowbound = (sizes * (1.0 - _EPS)).astype(jnp.int32)
    highbound = (sizes * (1.0 + _EPS)).astype(jnp.int32)
    return jnp.stack((lowbound, highbound))
